# baseline (speedup 1.0000x reference)
import jax
import jax.numpy as jnp
from jax.experimental import pallas as pl
from jax.experimental.pallas import tpu as pltpu

N = 10000
F = 128
S = 5
BM = 80
ROWS_PER_S = N // S
NBL = ROWS_PER_S // BM


def _elu(v):
    return jnp.where(v > 0, v, jnp.exp(jnp.minimum(v, 0.0)) - 1.0)


def _bf16_dot(a, b):
    return jnp.dot(
        a.astype(jnp.bfloat16),
        b.astype(jnp.bfloat16),
        preferred_element_type=jnp.float32,
    )


def _fused_kernel(*refs):
    a_refs = refs[:S]
    x_ref, w1_ref, b1_ref, wo_ref, bo_ref, o_ref, h0_ref, h1_ref = refs[S:]
    i = pl.program_id(0)

    @pl.when(i == 0)
    def _():
        h0_ref[:] = (_bf16_dot(x_ref[:], w1_ref[:]) + b1_ref[:]).astype(
            jnp.bfloat16
        )

    @pl.when(i < NBL)
    def _():
        for s in range(S):
            acc = jnp.dot(
                a_refs[s][:].astype(jnp.bfloat16),
                h0_ref[:],
                preferred_element_type=jnp.float32,
            )
            h1_ref[pl.ds(s * ROWS_PER_S + i * BM, BM), :] = _elu(acc).astype(
                jnp.bfloat16
            )

    @pl.when(i >= NBL)
    def _():
        for s in range(S):
            acc = jnp.dot(
                a_refs[s][:].astype(jnp.bfloat16),
                h1_ref[:],
                preferred_element_type=jnp.float32,
            )
            t = _elu(acc)
            o_ref[s] = _bf16_dot(t, wo_ref[:]) + bo_ref[:]


@jax.jit
def kernel(x, adjs, W1, b1, Wout, bout):
    const = lambda i: (0, 0)
    a_specs = [
        pl.BlockSpec(
            (BM, N), lambda i, s=s: (s * NBL + jax.lax.rem(i, NBL), 0)
        )
        for s in range(S)
    ]
    out = pl.pallas_call(
        _fused_kernel,
        grid=(2 * NBL,),
        in_specs=a_specs
        + [
            pl.BlockSpec((N, F), const),
            pl.BlockSpec((F, F), const),
            pl.BlockSpec((1, F), const),
            pl.BlockSpec((F, F), const),
            pl.BlockSpec((1, F), const),
        ],
        out_specs=pl.BlockSpec(
            (S, BM, F), lambda i: (0, jnp.maximum(i - NBL, 0), 0)
        ),
        out_shape=jax.ShapeDtypeStruct((S, ROWS_PER_S, F), jnp.float32),
        scratch_shapes=[
            pltpu.VMEM((N, F), jnp.bfloat16),
            pltpu.VMEM((N, F), jnp.bfloat16),
        ],
        compiler_params=pltpu.CompilerParams(
            vmem_limit_bytes=128 * 1024 * 1024,
        ),
    )(
        *([adjs] * S),
        x,
        W1.T,
        b1.reshape(1, F),
        Wout.T,
        bout.reshape(1, F),
    )
    return out.reshape(N, F)


# speedup vs baseline: 1.0170x; 1.0170x over previous
import jax
import jax.numpy as jnp
from jax.experimental import pallas as pl
from jax.experimental.pallas import tpu as pltpu

N = 10000
F = 128
S = 1
BM = 400
ROWS_PER_S = N // S
NBL = ROWS_PER_S // BM


def _elu(v):
    return jnp.where(v > 0, v, jnp.exp(jnp.minimum(v, 0.0)) - 1.0)


def _bf16_dot(a, b):
    return jnp.dot(
        a.astype(jnp.bfloat16),
        b.astype(jnp.bfloat16),
        preferred_element_type=jnp.float32,
    )


def _fused_kernel(*refs):
    a_refs = refs[:S]
    x_ref, w1_ref, b1_ref, wo_ref, bo_ref, o_ref, h0_ref, h1_ref = refs[S:]
    i = pl.program_id(0)

    @pl.when(i == 0)
    def _():
        h0_ref[:] = (_bf16_dot(x_ref[:], w1_ref[:]) + b1_ref[:]).astype(
            jnp.bfloat16
        )

    @pl.when(i < NBL)
    def _():
        for s in range(S):
            acc = jnp.dot(
                a_refs[s][:].astype(jnp.bfloat16),
                h0_ref[:],
                preferred_element_type=jnp.float32,
            )
            h1_ref[pl.ds(s * ROWS_PER_S + i * BM, BM), :] = _elu(acc).astype(
                jnp.bfloat16
            )

    @pl.when(i >= NBL)
    def _():
        for s in range(S):
            acc = jnp.dot(
                a_refs[s][:].astype(jnp.bfloat16),
                h1_ref[:],
                preferred_element_type=jnp.float32,
            )
            t = _elu(acc)
            o_ref[s] = _bf16_dot(t, wo_ref[:]) + bo_ref[:]


@jax.jit
def kernel(x, adjs, W1, b1, Wout, bout):
    const = lambda i: (0, 0)
    a_specs = [
        pl.BlockSpec(
            (BM, N), lambda i, s=s: (s * NBL + jax.lax.rem(i, NBL), 0)
        )
        for s in range(S)
    ]
    out = pl.pallas_call(
        _fused_kernel,
        grid=(2 * NBL,),
        in_specs=a_specs
        + [
            pl.BlockSpec((N, F), const),
            pl.BlockSpec((F, F), const),
            pl.BlockSpec((1, F), const),
            pl.BlockSpec((F, F), const),
            pl.BlockSpec((1, F), const),
        ],
        out_specs=pl.BlockSpec(
            (S, BM, F), lambda i: (0, jnp.maximum(i - NBL, 0), 0)
        ),
        out_shape=jax.ShapeDtypeStruct((S, ROWS_PER_S, F), jnp.float32),
        scratch_shapes=[
            pltpu.VMEM((N, F), jnp.bfloat16),
            pltpu.VMEM((N, F), jnp.bfloat16),
        ],
        compiler_params=pltpu.CompilerParams(
            vmem_limit_bytes=128 * 1024 * 1024,
        ),
    )(
        *([adjs] * S),
        x,
        W1.T,
        b1.reshape(1, F),
        Wout.T,
        bout.reshape(1, F),
    )
    return out.reshape(N, F)
